# baseline (speedup 1.0000x reference)
import jax
import jax.numpy as jnp
from jax import lax
from jax.experimental import pallas as pl
from jax.experimental.pallas import tpu as pltpu

_ALPHA = 0.2
_TR = 256
_CC = 512


def _elu(v):
    return jnp.where(v > 0, v, jnp.exp(v) - 1.0)


def _gat_body(x_ref, adj_ref, w0_ref, a0_ref, w1_ref, a1_ref, w2_ref, a2_ref,
              w3_ref, a3_ref, wout_ref, aout_ref, out_ref,
              wcat_scr, haug_scr, rowf_scr, colf_scr, xcat_scr, adj_scr,
              h2aug_scr, rowf2_scr, colf2_scr):
    l = pl.program_id(0)
    t = pl.program_id(1)
    n = xcat_scr.shape[0]
    nheads = rowf_scr.shape[1]
    f = h2aug_scr.shape[1] - 1
    fa = f + 1
    ncc = n // _CC
    w_refs = (w0_ref, w1_ref, w2_ref, w3_ref)
    a_refs = (a0_ref, a1_ref, a2_ref, a3_ref)

    @pl.when(jnp.logical_and(l == 0, t == 0))
    def _prep1():
        for hd in range(nheads):
            wcat_scr[:, hd * f:(hd + 1) * f] = w_refs[hd][...]
        h = jnp.dot(x_ref[...], wcat_scr[...],
                    preferred_element_type=jnp.float32)
        for hd in range(nheads):
            h_hd = h[:, hd * f:(hd + 1) * f]
            haug_scr[:, hd * fa:hd * fa + f] = h_hd.astype(jnp.bfloat16)
            haug_scr[:, hd * fa + f:(hd + 1) * fa] = jnp.ones(
                (n, 1), jnp.bfloat16)
            a = a_refs[hd][...]
            s_col = lax.dot_general(h_hd, a[:, :f], (((1,), (1,)), ((), ())),
                                    preferred_element_type=jnp.float32)
            dT = lax.dot_general(a[:, f:], h_hd, (((1,), (1,)), ((), ())),
                                 preferred_element_type=jnp.float32)
            rowf_scr[:, hd:hd + 1] = jnp.exp(
                (1.0 - _ALPHA) * s_col).astype(jnp.bfloat16)
            colf_scr[hd:hd + 1, :] = jnp.exp(-dT).astype(jnp.bfloat16)
            colf_scr[nheads + hd:nheads + hd + 1, :] = jnp.exp(
                -_ALPHA * dT).astype(jnp.bfloat16)

    @pl.when(l == 0)
    def _layer1():
        rows = pl.ds(t * _TR, _TR)
        adj = adj_ref[...].astype(jnp.bfloat16)
        adj_scr[rows, :] = adj
        rowf = rowf_scr[rows, :]
        for hd in range(nheads):
            r_bc = jnp.broadcast_to(rowf[:, hd:hd + 1], (_TR, _CC))
            acc = jnp.zeros((_TR, fa), jnp.float32)
            for c in range(ncc):
                cols = slice(c * _CC, (c + 1) * _CC)
                b_row = colf_scr[hd:hd + 1, cols]
                d_row = colf_scr[nheads + hd:nheads + hd + 1, cols]
                e = jnp.minimum(b_row, r_bc * d_row) * adj[:, cols]
                acc += jnp.dot(e, haug_scr[cols, hd * fa:(hd + 1) * fa],
                               preferred_element_type=jnp.float32)
            xcat_scr[rows, hd * f:(hd + 1) * f] = _elu(
                acc[:, :f] / acc[:, f:fa])
        out_ref[...] = jnp.zeros_like(out_ref)

    @pl.when(jnp.logical_and(l == 1, t == 0))
    def _prep2():
        h2 = jnp.dot(xcat_scr[...], wout_ref[...],
                     preferred_element_type=jnp.float32)
        h2aug_scr[:, :f] = h2.astype(jnp.bfloat16)
        h2aug_scr[:, f:] = jnp.ones((n, 1), jnp.bfloat16)
        a2 = aout_ref[...]
        s2 = lax.dot_general(h2, a2[:, :f], (((1,), (1,)), ((), ())),
                             preferred_element_type=jnp.float32)
        d2T = lax.dot_general(a2[:, f:], h2, (((1,), (1,)), ((), ())),
                              preferred_element_type=jnp.float32)
        rowf2_scr[...] = jnp.exp((1.0 - _ALPHA) * s2).astype(jnp.bfloat16)
        colf2_scr[0:1, :] = jnp.exp(-d2T).astype(jnp.bfloat16)
        colf2_scr[1:2, :] = jnp.exp(-_ALPHA * d2T).astype(jnp.bfloat16)

    @pl.when(l == 1)
    def _layer2():
        rows = pl.ds(t * _TR, _TR)
        r_bc = jnp.broadcast_to(rowf2_scr[rows, 0:1], (_TR, _CC))
        acc = jnp.zeros((_TR, fa), jnp.float32)
        for c in range(ncc):
            cols = slice(c * _CC, (c + 1) * _CC)
            b_row = colf2_scr[0:1, cols]
            d_row = colf2_scr[1:2, cols]
            e = jnp.minimum(b_row, r_bc * d_row) * adj_scr[rows, cols]
            acc += jnp.dot(e, h2aug_scr[cols, :],
                           preferred_element_type=jnp.float32)
        out_ref[...] = _elu(acc[:, :f] / acc[:, f:fa])


def kernel(x, adj, W0, a0, W1, a1, W2, a2, W3, a3, W_out, a_out):
    n, nfeat = x.shape
    f = W0.shape[1]
    nheads = 4
    fcat = nheads * f
    nt = n // _TR

    const = lambda l, t: (0, 0)
    wspec = pl.BlockSpec((nfeat, f), const)
    aspec = pl.BlockSpec((1, 2 * f), const)
    grid = (2, nt)
    return pl.pallas_call(
        _gat_body,
        grid=grid,
        in_specs=[
            pl.BlockSpec((n, nfeat), const),
            pl.BlockSpec((_TR, n),
                         lambda l, t: (jnp.where(l == 0, t, nt - 1), 0)),
            wspec, aspec, wspec, aspec, wspec, aspec, wspec, aspec,
            pl.BlockSpec((fcat, f), const),
            aspec,
        ],
        out_specs=pl.BlockSpec((_TR, f), lambda l, t: (t, 0)),
        out_shape=jax.ShapeDtypeStruct((n, f), jnp.float32),
        scratch_shapes=[
            pltpu.VMEM((nfeat, fcat), jnp.float32),
            pltpu.VMEM((n, nheads * (f + 1)), jnp.bfloat16),
            pltpu.VMEM((n, nheads), jnp.bfloat16),
            pltpu.VMEM((2 * nheads, n), jnp.bfloat16),
            pltpu.VMEM((n, fcat), jnp.float32),
            pltpu.VMEM((n, n), jnp.bfloat16),
            pltpu.VMEM((n, f + 1), jnp.bfloat16),
            pltpu.VMEM((n, 1), jnp.bfloat16),
            pltpu.VMEM((2, n), jnp.bfloat16),
        ],
        compiler_params=pltpu.CompilerParams(
            dimension_semantics=("arbitrary", "arbitrary")),
    )(x, adj, W0, a0, W1, a1, W2, a2, W3, a3, W_out, a_out)


# speedup vs baseline: 1.1489x; 1.1489x over previous
import jax
import jax.numpy as jnp
from jax import lax
from jax.experimental import pallas as pl
from jax.experimental.pallas import tpu as pltpu

_ALPHA = 0.2
_TR = 256
_CC = 256


def _elu(v):
    return jnp.where(v > 0, v, jnp.exp(v) - 1.0)


def _gat_body(x_ref, adj_ref, w0_ref, a0_ref, w1_ref, a1_ref, w2_ref, a2_ref,
              w3_ref, a3_ref, wout_ref, aout_ref, out_ref,
              wcat_scr, aT_scr, haug_scr, rowf_scr, colf_scr, xcat_scr,
              adj_scr, h2aug_scr, rowf2_scr, colf2_scr):
    l = pl.program_id(0)
    t = pl.program_id(1)
    n = xcat_scr.shape[0]
    nheads = rowf_scr.shape[1]
    f = h2aug_scr.shape[1] - 1
    fa = f + 1
    ncc = n // _CC
    w_refs = (w0_ref, w1_ref, w2_ref, w3_ref)
    a_refs = (a0_ref, a1_ref, a2_ref, a3_ref)

    @pl.when(jnp.logical_and(l == 0, t == 0))
    def _prep1():
        aT_scr[...] = jnp.zeros_like(aT_scr)
        for hd in range(nheads):
            wcat_scr[:, hd * f:(hd + 1) * f] = w_refs[hd][...]
            a = a_refs[hd][...]
            aT_scr[hd:hd + 1, hd * f:(hd + 1) * f] = a[:, :f]
            aT_scr[nheads + hd:nheads + hd + 1, hd * f:(hd + 1) * f] = a[:, f:]
        h = jnp.dot(x_ref[...], wcat_scr[...],
                    preferred_element_type=jnp.float32)
        for hd in range(nheads):
            haug_scr[:, hd * fa:hd * fa + f] = (
                h[:, hd * f:(hd + 1) * f].astype(jnp.bfloat16))
            haug_scr[:, hd * fa + f:(hd + 1) * fa] = jnp.ones(
                (n, 1), jnp.bfloat16)
        s_cols = lax.dot_general(h, aT_scr[:nheads, :],
                                 (((1,), (1,)), ((), ())),
                                 preferred_element_type=jnp.float32)
        dT = lax.dot_general(aT_scr[nheads:, :], h, (((1,), (1,)), ((), ())),
                             preferred_element_type=jnp.float32)
        rowf_scr[...] = jnp.exp((1.0 - _ALPHA) * s_cols).astype(jnp.bfloat16)
        colf_scr[:nheads, :] = jnp.exp(-dT).astype(jnp.bfloat16)
        colf_scr[nheads:, :] = jnp.exp(-_ALPHA * dT).astype(jnp.bfloat16)

    @pl.when(l == 0)
    def _layer1():
        rows = pl.ds(t * _TR, _TR)
        adj = adj_ref[...].astype(jnp.bfloat16)
        adj_scr[rows, :] = adj
        rowf = rowf_scr[rows, :]
        for hd in range(nheads):
            r_bc = jnp.broadcast_to(rowf[:, hd:hd + 1], (_TR, _CC))
            acc = jnp.zeros((_TR, fa), jnp.float32)
            for c in range(ncc):
                cols = slice(c * _CC, (c + 1) * _CC)
                b_row = colf_scr[hd:hd + 1, cols]
                d_row = colf_scr[nheads + hd:nheads + hd + 1, cols]
                e = jnp.minimum(b_row, r_bc * d_row) * adj[:, cols]
                acc += jnp.dot(e, haug_scr[cols, hd * fa:(hd + 1) * fa],
                               preferred_element_type=jnp.float32)
            xcat_scr[rows, hd * f:(hd + 1) * f] = _elu(
                acc[:, :f] / acc[:, f:fa])
        out_ref[...] = jnp.zeros_like(out_ref)

    @pl.when(jnp.logical_and(l == 1, t == 0))
    def _prep2():
        h2 = jnp.dot(xcat_scr[...], wout_ref[...],
                     preferred_element_type=jnp.float32)
        h2aug_scr[:, :f] = h2.astype(jnp.bfloat16)
        h2aug_scr[:, f:] = jnp.ones((n, 1), jnp.bfloat16)
        a2 = aout_ref[...]
        s2 = lax.dot_general(h2, a2[:, :f], (((1,), (1,)), ((), ())),
                             preferred_element_type=jnp.float32)
        d2T = lax.dot_general(a2[:, f:], h2, (((1,), (1,)), ((), ())),
                              preferred_element_type=jnp.float32)
        rowf2_scr[...] = jnp.exp((1.0 - _ALPHA) * s2).astype(jnp.bfloat16)
        colf2_scr[0:1, :] = jnp.exp(-d2T).astype(jnp.bfloat16)
        colf2_scr[1:2, :] = jnp.exp(-_ALPHA * d2T).astype(jnp.bfloat16)

    @pl.when(l == 1)
    def _layer2():
        rows = pl.ds(t * _TR, _TR)
        r_bc = jnp.broadcast_to(rowf2_scr[rows, 0:1], (_TR, _CC))
        acc = jnp.zeros((_TR, fa), jnp.float32)
        for c in range(ncc):
            cols = slice(c * _CC, (c + 1) * _CC)
            b_row = colf2_scr[0:1, cols]
            d_row = colf2_scr[1:2, cols]
            e = jnp.minimum(b_row, r_bc * d_row) * adj_scr[rows, cols]
            acc += jnp.dot(e, h2aug_scr[cols, :],
                           preferred_element_type=jnp.float32)
        out_ref[...] = _elu(acc[:, :f] / acc[:, f:fa])


def kernel(x, adj, W0, a0, W1, a1, W2, a2, W3, a3, W_out, a_out):
    n, nfeat = x.shape
    f = W0.shape[1]
    nheads = 4
    fcat = nheads * f
    nt = n // _TR

    const = lambda l, t: (0, 0)
    wspec = pl.BlockSpec((nfeat, f), const)
    aspec = pl.BlockSpec((1, 2 * f), const)
    grid = (2, nt)
    return pl.pallas_call(
        _gat_body,
        grid=grid,
        in_specs=[
            pl.BlockSpec((n, nfeat), const),
            pl.BlockSpec((_TR, n),
                         lambda l, t: (jnp.where(l == 0, t, nt - 1), 0)),
            wspec, aspec, wspec, aspec, wspec, aspec, wspec, aspec,
            pl.BlockSpec((fcat, f), const),
            aspec,
        ],
        out_specs=pl.BlockSpec((_TR, f), lambda l, t: (t, 0)),
        out_shape=jax.ShapeDtypeStruct((n, f), jnp.float32),
        scratch_shapes=[
            pltpu.VMEM((nfeat, fcat), jnp.float32),
            pltpu.VMEM((2 * nheads, fcat), jnp.float32),
            pltpu.VMEM((n, nheads * (f + 1)), jnp.bfloat16),
            pltpu.VMEM((n, nheads), jnp.bfloat16),
            pltpu.VMEM((2 * nheads, n), jnp.bfloat16),
            pltpu.VMEM((n, fcat), jnp.float32),
            pltpu.VMEM((n, n), jnp.bfloat16),
            pltpu.VMEM((n, f + 1), jnp.bfloat16),
            pltpu.VMEM((n, 1), jnp.bfloat16),
            pltpu.VMEM((2, n), jnp.bfloat16),
        ],
        compiler_params=pltpu.CompilerParams(
            dimension_semantics=("arbitrary", "arbitrary")),
    )(x, adj, W0, a0, W1, a1, W2, a2, W3, a3, W_out, a_out)


# speedup vs baseline: 1.1527x; 1.0033x over previous
import jax
import jax.numpy as jnp
from jax import lax
from jax.experimental import pallas as pl
from jax.experimental.pallas import tpu as pltpu

_ALPHA = 0.2
_TR = 256
_CC = 2048


def _elu(v):
    return jnp.where(v > 0, v, jnp.exp(v) - 1.0)


def _gat_body(x_ref, adj_ref, w0_ref, a0_ref, w1_ref, a1_ref, w2_ref, a2_ref,
              w3_ref, a3_ref, wout_ref, aout_ref, out_ref,
              wcat_scr, aT_scr, haug_scr, rowf_scr, colf_scr, xcat_scr,
              adj_scr, h2aug_scr, rowf2_scr, colf2_scr):
    l = pl.program_id(0)
    t = pl.program_id(1)
    n = xcat_scr.shape[0]
    nheads = rowf_scr.shape[1]
    f = h2aug_scr.shape[1] - 1
    fa = f + 1
    ncc = n // _CC
    w_refs = (w0_ref, w1_ref, w2_ref, w3_ref)
    a_refs = (a0_ref, a1_ref, a2_ref, a3_ref)

    @pl.when(jnp.logical_and(l == 0, t == 0))
    def _prep1():
        aT_scr[...] = jnp.zeros_like(aT_scr)
        for hd in range(nheads):
            wcat_scr[:, hd * f:(hd + 1) * f] = w_refs[hd][...]
            a = a_refs[hd][...]
            aT_scr[hd:hd + 1, hd * f:(hd + 1) * f] = a[:, :f]
            aT_scr[nheads + hd:nheads + hd + 1, hd * f:(hd + 1) * f] = a[:, f:]
        h = jnp.dot(x_ref[...], wcat_scr[...],
                    preferred_element_type=jnp.float32)
        for hd in range(nheads):
            haug_scr[:, hd * fa:hd * fa + f] = (
                h[:, hd * f:(hd + 1) * f].astype(jnp.bfloat16))
            haug_scr[:, hd * fa + f:(hd + 1) * fa] = jnp.ones(
                (n, 1), jnp.bfloat16)
        s_cols = lax.dot_general(h, aT_scr[:nheads, :],
                                 (((1,), (1,)), ((), ())),
                                 preferred_element_type=jnp.float32)
        dT = lax.dot_general(aT_scr[nheads:, :], h, (((1,), (1,)), ((), ())),
                             preferred_element_type=jnp.float32)
        rowf_scr[...] = jnp.exp((1.0 - _ALPHA) * s_cols).astype(jnp.bfloat16)
        colf_scr[:nheads, :] = jnp.exp(-dT).astype(jnp.bfloat16)
        colf_scr[nheads:, :] = jnp.exp(-_ALPHA * dT).astype(jnp.bfloat16)

    @pl.when(l == 0)
    def _layer1():
        rows = pl.ds(t * _TR, _TR)
        adj = adj_ref[...].astype(jnp.bfloat16)
        adj_scr[rows, :] = adj
        rowf = rowf_scr[rows, :]
        for hd in range(nheads):
            r_bc = jnp.broadcast_to(rowf[:, hd:hd + 1], (_TR, _CC))
            acc = jnp.zeros((_TR, fa), jnp.float32)
            for c in range(ncc):
                cols = slice(c * _CC, (c + 1) * _CC)
                b_row = colf_scr[hd:hd + 1, cols]
                d_row = colf_scr[nheads + hd:nheads + hd + 1, cols]
                e = jnp.minimum(b_row, r_bc * d_row) * adj[:, cols]
                acc += jnp.dot(e, haug_scr[cols, hd * fa:(hd + 1) * fa],
                               preferred_element_type=jnp.float32)
            xcat_scr[rows, hd * f:(hd + 1) * f] = _elu(
                acc[:, :f] / acc[:, f:fa])
        out_ref[...] = jnp.zeros_like(out_ref)

    @pl.when(jnp.logical_and(l == 1, t == 0))
    def _prep2():
        h2 = jnp.dot(xcat_scr[...], wout_ref[...],
                     preferred_element_type=jnp.float32)
        h2aug_scr[:, :f] = h2.astype(jnp.bfloat16)
        h2aug_scr[:, f:] = jnp.ones((n, 1), jnp.bfloat16)
        a2 = aout_ref[...]
        s2 = lax.dot_general(h2, a2[:, :f], (((1,), (1,)), ((), ())),
                             preferred_element_type=jnp.float32)
        d2T = lax.dot_general(a2[:, f:], h2, (((1,), (1,)), ((), ())),
                              preferred_element_type=jnp.float32)
        rowf2_scr[...] = jnp.exp((1.0 - _ALPHA) * s2).astype(jnp.bfloat16)
        colf2_scr[0:1, :] = jnp.exp(-d2T).astype(jnp.bfloat16)
        colf2_scr[1:2, :] = jnp.exp(-_ALPHA * d2T).astype(jnp.bfloat16)

    @pl.when(l == 1)
    def _layer2():
        rows = pl.ds(t * _TR, _TR)
        r_bc = jnp.broadcast_to(rowf2_scr[rows, 0:1], (_TR, _CC))
        acc = jnp.zeros((_TR, fa), jnp.float32)
        for c in range(ncc):
            cols = slice(c * _CC, (c + 1) * _CC)
            b_row = colf2_scr[0:1, cols]
            d_row = colf2_scr[1:2, cols]
            e = jnp.minimum(b_row, r_bc * d_row) * adj_scr[rows, cols]
            acc += jnp.dot(e, h2aug_scr[cols, :],
                           preferred_element_type=jnp.float32)
        out_ref[...] = _elu(acc[:, :f] / acc[:, f:fa])


def kernel(x, adj, W0, a0, W1, a1, W2, a2, W3, a3, W_out, a_out):
    n, nfeat = x.shape
    f = W0.shape[1]
    nheads = 4
    fcat = nheads * f
    nt = n // _TR

    const = lambda l, t: (0, 0)
    wspec = pl.BlockSpec((nfeat, f), const)
    aspec = pl.BlockSpec((1, 2 * f), const)
    grid = (2, nt)
    return pl.pallas_call(
        _gat_body,
        grid=grid,
        in_specs=[
            pl.BlockSpec((n, nfeat), const),
            pl.BlockSpec((_TR, n),
                         lambda l, t: (jnp.where(l == 0, t, nt - 1), 0)),
            wspec, aspec, wspec, aspec, wspec, aspec, wspec, aspec,
            pl.BlockSpec((fcat, f), const),
            aspec,
        ],
        out_specs=pl.BlockSpec((_TR, f), lambda l, t: (t, 0)),
        out_shape=jax.ShapeDtypeStruct((n, f), jnp.float32),
        scratch_shapes=[
            pltpu.VMEM((nfeat, fcat), jnp.float32),
            pltpu.VMEM((2 * nheads, fcat), jnp.float32),
            pltpu.VMEM((n, nheads * (f + 1)), jnp.bfloat16),
            pltpu.VMEM((n, nheads), jnp.bfloat16),
            pltpu.VMEM((2 * nheads, n), jnp.bfloat16),
            pltpu.VMEM((n, fcat), jnp.float32),
            pltpu.VMEM((n, n), jnp.bfloat16),
            pltpu.VMEM((n, f + 1), jnp.bfloat16),
            pltpu.VMEM((n, 1), jnp.bfloat16),
            pltpu.VMEM((2, n), jnp.bfloat16),
        ],
        compiler_params=pltpu.CompilerParams(
            dimension_semantics=("arbitrary", "arbitrary")),
    )(x, adj, W0, a0, W1, a1, W2, a2, W3, a3, W_out, a_out)


# speedup vs baseline: 1.2751x; 1.1061x over previous
import jax
import jax.numpy as jnp
from jax import lax
from jax.experimental import pallas as pl
from jax.experimental.pallas import tpu as pltpu

_ALPHA = 0.2
_TR = 512
_CC = 2048


def _elu(v):
    return jnp.where(v > 0, v, jnp.exp(v) - 1.0)


def _gat_body(x_ref, adj_ref, w0_ref, a0_ref, w1_ref, a1_ref, w2_ref, a2_ref,
              w3_ref, a3_ref, wout_ref, aout_ref, out_ref,
              wcat_scr, aT_scr, haug_scr, rowf_scr, colf_scr, xcat_scr,
              adj_scr, h2aug_scr, rowf2_scr, colf2_scr):
    l = pl.program_id(0)
    t = pl.program_id(1)
    n = xcat_scr.shape[0]
    nheads = rowf_scr.shape[1]
    f = h2aug_scr.shape[1] - 1
    fa = f + 1
    ncc = n // _CC
    w_refs = (w0_ref, w1_ref, w2_ref, w3_ref)
    a_refs = (a0_ref, a1_ref, a2_ref, a3_ref)

    @pl.when(jnp.logical_and(l == 0, t == 0))
    def _prep1():
        aT_scr[...] = jnp.zeros_like(aT_scr)
        for hd in range(nheads):
            wcat_scr[:, hd * f:(hd + 1) * f] = w_refs[hd][...]
            a = a_refs[hd][...]
            aT_scr[hd:hd + 1, hd * f:(hd + 1) * f] = a[:, :f]
            aT_scr[nheads + hd:nheads + hd + 1, hd * f:(hd + 1) * f] = a[:, f:]
        h = jnp.dot(x_ref[...], wcat_scr[...],
                    preferred_element_type=jnp.float32)
        for hd in range(nheads):
            haug_scr[:, hd * fa:hd * fa + f] = (
                h[:, hd * f:(hd + 1) * f].astype(jnp.bfloat16))
            haug_scr[:, hd * fa + f:(hd + 1) * fa] = jnp.ones(
                (n, 1), jnp.bfloat16)
        s_cols = lax.dot_general(h, aT_scr[:nheads, :],
                                 (((1,), (1,)), ((), ())),
                                 preferred_element_type=jnp.float32)
        dT = lax.dot_general(aT_scr[nheads:, :], h, (((1,), (1,)), ((), ())),
                             preferred_element_type=jnp.float32)
        rowf_scr[...] = jnp.exp((1.0 - _ALPHA) * s_cols).astype(jnp.bfloat16)
        colf_scr[:nheads, :] = jnp.exp(-dT).astype(jnp.bfloat16)
        colf_scr[nheads:, :] = jnp.exp(-_ALPHA * dT).astype(jnp.bfloat16)

    @pl.when(l == 0)
    def _layer1():
        rows = pl.ds(t * _TR, _TR)
        adj = adj_ref[...].astype(jnp.bfloat16)
        adj_scr[rows, :] = adj
        rowf = rowf_scr[rows, :]
        for hd in range(nheads):
            r_bc = jnp.broadcast_to(rowf[:, hd:hd + 1], (_TR, _CC))
            acc = jnp.zeros((_TR, fa), jnp.float32)
            for c in range(ncc):
                cols = slice(c * _CC, (c + 1) * _CC)
                b_row = colf_scr[hd:hd + 1, cols]
                d_row = colf_scr[nheads + hd:nheads + hd + 1, cols]
                e = jnp.minimum(b_row, r_bc * d_row) * adj[:, cols]
                acc += jnp.dot(e, haug_scr[cols, hd * fa:(hd + 1) * fa],
                               preferred_element_type=jnp.float32)
            xcat_scr[rows, hd * f:(hd + 1) * f] = _elu(
                acc[:, :f] / acc[:, f:fa])
        out_ref[...] = jnp.zeros_like(out_ref)

    @pl.when(jnp.logical_and(l == 1, t == 0))
    def _prep2():
        h2 = jnp.dot(xcat_scr[...], wout_ref[...],
                     preferred_element_type=jnp.float32)
        h2aug_scr[:, :f] = h2.astype(jnp.bfloat16)
        h2aug_scr[:, f:] = jnp.ones((n, 1), jnp.bfloat16)
        a2 = aout_ref[...]
        s2 = lax.dot_general(h2, a2[:, :f], (((1,), (1,)), ((), ())),
                             preferred_element_type=jnp.float32)
        d2T = lax.dot_general(a2[:, f:], h2, (((1,), (1,)), ((), ())),
                              preferred_element_type=jnp.float32)
        rowf2_scr[...] = jnp.exp((1.0 - _ALPHA) * s2).astype(jnp.bfloat16)
        colf2_scr[0:1, :] = jnp.exp(-d2T).astype(jnp.bfloat16)
        colf2_scr[1:2, :] = jnp.exp(-_ALPHA * d2T).astype(jnp.bfloat16)

    @pl.when(l == 1)
    def _layer2():
        rows = pl.ds(t * _TR, _TR)
        r_bc = jnp.broadcast_to(rowf2_scr[rows, 0:1], (_TR, _CC))
        acc = jnp.zeros((_TR, fa), jnp.float32)
        for c in range(ncc):
            cols = slice(c * _CC, (c + 1) * _CC)
            b_row = colf2_scr[0:1, cols]
            d_row = colf2_scr[1:2, cols]
            e = jnp.minimum(b_row, r_bc * d_row) * adj_scr[rows, cols]
            acc += jnp.dot(e, h2aug_scr[cols, :],
                           preferred_element_type=jnp.float32)
        out_ref[...] = _elu(acc[:, :f] / acc[:, f:fa])


def kernel(x, adj, W0, a0, W1, a1, W2, a2, W3, a3, W_out, a_out):
    n, nfeat = x.shape
    f = W0.shape[1]
    nheads = 4
    fcat = nheads * f
    nt = n // _TR

    const = lambda l, t: (0, 0)
    wspec = pl.BlockSpec((nfeat, f), const)
    aspec = pl.BlockSpec((1, 2 * f), const)
    grid = (2, nt)
    return pl.pallas_call(
        _gat_body,
        grid=grid,
        in_specs=[
            pl.BlockSpec((n, nfeat), const),
            pl.BlockSpec((_TR, n),
                         lambda l, t: (jnp.where(l == 0, t, nt - 1), 0)),
            wspec, aspec, wspec, aspec, wspec, aspec, wspec, aspec,
            pl.BlockSpec((fcat, f), const),
            aspec,
        ],
        out_specs=pl.BlockSpec((_TR, f), lambda l, t: (t, 0)),
        out_shape=jax.ShapeDtypeStruct((n, f), jnp.float32),
        scratch_shapes=[
            pltpu.VMEM((nfeat, fcat), jnp.float32),
            pltpu.VMEM((2 * nheads, fcat), jnp.float32),
            pltpu.VMEM((n, nheads * (f + 1)), jnp.bfloat16),
            pltpu.VMEM((n, nheads), jnp.bfloat16),
            pltpu.VMEM((2 * nheads, n), jnp.bfloat16),
            pltpu.VMEM((n, fcat), jnp.float32),
            pltpu.VMEM((n, n), jnp.bfloat16),
            pltpu.VMEM((n, f + 1), jnp.bfloat16),
            pltpu.VMEM((n, 1), jnp.bfloat16),
            pltpu.VMEM((2, n), jnp.bfloat16),
        ],
        compiler_params=pltpu.CompilerParams(
            dimension_semantics=("arbitrary", "arbitrary")),
    )(x, adj, W0, a0, W1, a1, W2, a2, W3, a3, W_out, a_out)


# speedup vs baseline: 1.2886x; 1.0106x over previous
import jax
import jax.numpy as jnp
from jax import lax
from jax.experimental import pallas as pl
from jax.experimental.pallas import tpu as pltpu

_ALPHA = 0.2
_TR = 1024
_CC = 2048


def _elu(v):
    return jnp.where(v > 0, v, jnp.exp(v) - 1.0)


def _gat_body(x_ref, adj_ref, w0_ref, a0_ref, w1_ref, a1_ref, w2_ref, a2_ref,
              w3_ref, a3_ref, wout_ref, aout_ref, out_ref,
              wcat_scr, aT_scr, haug_scr, rowf_scr, colf_scr, xcat_scr,
              adj_scr, h2aug_scr, rowf2_scr, colf2_scr):
    l = pl.program_id(0)
    t = pl.program_id(1)
    n = xcat_scr.shape[0]
    nheads = rowf_scr.shape[1]
    f = h2aug_scr.shape[1] - 1
    fa = f + 1
    ncc = n // _CC
    w_refs = (w0_ref, w1_ref, w2_ref, w3_ref)
    a_refs = (a0_ref, a1_ref, a2_ref, a3_ref)

    @pl.when(jnp.logical_and(l == 0, t == 0))
    def _prep1():
        aT_scr[...] = jnp.zeros_like(aT_scr)
        for hd in range(nheads):
            wcat_scr[:, hd * f:(hd + 1) * f] = w_refs[hd][...]
            a = a_refs[hd][...]
            aT_scr[hd:hd + 1, hd * f:(hd + 1) * f] = a[:, :f]
            aT_scr[nheads + hd:nheads + hd + 1, hd * f:(hd + 1) * f] = a[:, f:]
        h = jnp.dot(x_ref[...], wcat_scr[...],
                    preferred_element_type=jnp.float32)
        for hd in range(nheads):
            haug_scr[:, hd * fa:hd * fa + f] = (
                h[:, hd * f:(hd + 1) * f].astype(jnp.bfloat16))
            haug_scr[:, hd * fa + f:(hd + 1) * fa] = jnp.ones(
                (n, 1), jnp.bfloat16)
        s_cols = lax.dot_general(h, aT_scr[:nheads, :],
                                 (((1,), (1,)), ((), ())),
                                 preferred_element_type=jnp.float32)
        dT = lax.dot_general(aT_scr[nheads:, :], h, (((1,), (1,)), ((), ())),
                             preferred_element_type=jnp.float32)
        rowf_scr[...] = jnp.exp((1.0 - _ALPHA) * s_cols).astype(jnp.bfloat16)
        colf_scr[:nheads, :] = jnp.exp(-dT).astype(jnp.bfloat16)
        colf_scr[nheads:, :] = jnp.exp(-_ALPHA * dT).astype(jnp.bfloat16)

    @pl.when(l == 0)
    def _layer1():
        rows = pl.ds(t * _TR, _TR)
        adj = adj_ref[...].astype(jnp.bfloat16)
        adj_scr[rows, :] = adj
        rowf = rowf_scr[rows, :]
        for hd in range(nheads):
            r_bc = jnp.broadcast_to(rowf[:, hd:hd + 1], (_TR, _CC))
            acc = jnp.zeros((_TR, fa), jnp.float32)
            for c in range(ncc):
                cols = slice(c * _CC, (c + 1) * _CC)
                b_row = colf_scr[hd:hd + 1, cols]
                d_row = colf_scr[nheads + hd:nheads + hd + 1, cols]
                e = jnp.minimum(b_row, r_bc * d_row) * adj[:, cols]
                acc += jnp.dot(e, haug_scr[cols, hd * fa:(hd + 1) * fa],
                               preferred_element_type=jnp.float32)
            xcat_scr[rows, hd * f:(hd + 1) * f] = _elu(
                acc[:, :f] / acc[:, f:fa])
        out_ref[...] = jnp.zeros_like(out_ref)

    @pl.when(jnp.logical_and(l == 1, t == 0))
    def _prep2():
        h2 = jnp.dot(xcat_scr[...], wout_ref[...],
                     preferred_element_type=jnp.float32)
        h2aug_scr[:, :f] = h2.astype(jnp.bfloat16)
        h2aug_scr[:, f:] = jnp.ones((n, 1), jnp.bfloat16)
        a2 = aout_ref[...]
        s2 = lax.dot_general(h2, a2[:, :f], (((1,), (1,)), ((), ())),
                             preferred_element_type=jnp.float32)
        d2T = lax.dot_general(a2[:, f:], h2, (((1,), (1,)), ((), ())),
                              preferred_element_type=jnp.float32)
        rowf2_scr[...] = jnp.exp((1.0 - _ALPHA) * s2).astype(jnp.bfloat16)
        colf2_scr[0:1, :] = jnp.exp(-d2T).astype(jnp.bfloat16)
        colf2_scr[1:2, :] = jnp.exp(-_ALPHA * d2T).astype(jnp.bfloat16)

    @pl.when(l == 1)
    def _layer2():
        rows = pl.ds(t * _TR, _TR)
        r_bc = jnp.broadcast_to(rowf2_scr[rows, 0:1], (_TR, _CC))
        acc = jnp.zeros((_TR, fa), jnp.float32)
        for c in range(ncc):
            cols = slice(c * _CC, (c + 1) * _CC)
            b_row = colf2_scr[0:1, cols]
            d_row = colf2_scr[1:2, cols]
            e = jnp.minimum(b_row, r_bc * d_row) * adj_scr[rows, cols]
            acc += jnp.dot(e, h2aug_scr[cols, :],
                           preferred_element_type=jnp.float32)
        out_ref[...] = _elu(acc[:, :f] / acc[:, f:fa])


def kernel(x, adj, W0, a0, W1, a1, W2, a2, W3, a3, W_out, a_out):
    n, nfeat = x.shape
    f = W0.shape[1]
    nheads = 4
    fcat = nheads * f
    nt = n // _TR

    const = lambda l, t: (0, 0)
    wspec = pl.BlockSpec((nfeat, f), const)
    aspec = pl.BlockSpec((1, 2 * f), const)
    grid = (2, nt)
    return pl.pallas_call(
        _gat_body,
        grid=grid,
        in_specs=[
            pl.BlockSpec((n, nfeat), const),
            pl.BlockSpec((_TR, n),
                         lambda l, t: (jnp.where(l == 0, t, nt - 1), 0)),
            wspec, aspec, wspec, aspec, wspec, aspec, wspec, aspec,
            pl.BlockSpec((fcat, f), const),
            aspec,
        ],
        out_specs=pl.BlockSpec((_TR, f), lambda l, t: (t, 0)),
        out_shape=jax.ShapeDtypeStruct((n, f), jnp.float32),
        scratch_shapes=[
            pltpu.VMEM((nfeat, fcat), jnp.float32),
            pltpu.VMEM((2 * nheads, fcat), jnp.float32),
            pltpu.VMEM((n, nheads * (f + 1)), jnp.bfloat16),
            pltpu.VMEM((n, nheads), jnp.bfloat16),
            pltpu.VMEM((2 * nheads, n), jnp.bfloat16),
            pltpu.VMEM((n, fcat), jnp.float32),
            pltpu.VMEM((n, n), jnp.bfloat16),
            pltpu.VMEM((n, f + 1), jnp.bfloat16),
            pltpu.VMEM((n, 1), jnp.bfloat16),
            pltpu.VMEM((2, n), jnp.bfloat16),
        ],
        compiler_params=pltpu.CompilerParams(
            dimension_semantics=("arbitrary", "arbitrary")),
    )(x, adj, W0, a0, W1, a1, W2, a2, W3, a3, W_out, a_out)
